# baseline (speedup 1.0000x reference)
import functools

import jax
import jax.numpy as jnp
from jax import lax
from jax.experimental import pallas as pl
from jax.experimental.pallas import tpu as pltpu
from jax.experimental.pallas import tpu_sc as plsc

HIDDEN = 1024
EMB = 128
BATCH = 4096
NUM_CAND = 200

NUM_CORES = 2
NUM_SUBCORES = 16
NUM_WORKERS = NUM_CORES * NUM_SUBCORES
RPW = BATCH // NUM_WORKERS

CHUNKS = ((0, 104), (104, 24), (128, 72))
COL0 = 128
COL1 = NUM_CAND - COL0

NUM_GROUPS = -(-NUM_CAND // 16)
CAND_PAD = NUM_GROUPS * 16

NBUF = 3


def _proj_body(p_ref, w_ref, b_ref, o_ref):
    o_ref[...] = (
        jnp.dot(p_ref[...], w_ref[...], preferred_element_type=jnp.float32)
        + b_ref[...]
    )


def _project(pooled, W, b):
    BB = 512
    return pl.pallas_call(
        _proj_body,
        grid=(BATCH // BB,),
        in_specs=[
            pl.BlockSpec((BB, HIDDEN), lambda i: (i, 0)),
            pl.BlockSpec((HIDDEN, EMB), lambda i: (0, 0)),
            pl.BlockSpec((EMB,), lambda i: (0,)),
        ],
        out_specs=pl.BlockSpec((BB, EMB), lambda i: (i, 0)),
        out_shape=jax.ShapeDtypeStruct((BATCH, EMB), jnp.float32),
    )(pooled, W, b)


def _sc_body(
    table, cand, text, out,
    idx_a, idx_b, text_all, rows_0, rows_1, rows_2, st_0, st_1, st_2,
    sem_0, sem_1, sem_2, osem_0, osem_1, osem_2, sem_in, sem_tx,
):
    cid = lax.axis_index("c")
    sid = lax.axis_index("s")
    wid = sid * NUM_CORES + cid
    row0 = wid * RPW

    bufs = [rows_0, rows_1, rows_2]
    stagings = [st_0, st_1, st_2]
    gsems = [sem_0, sem_1, sem_2]
    osems = [osem_0, osem_1, osem_2]

    pltpu.async_copy(
        cand.at[pl.ds(row0, RPW), pl.ds(0, COL0)], idx_a, sem_in
    )
    pltpu.async_copy(
        cand.at[pl.ds(row0, RPW), pl.ds(COL0, COL1)], idx_b, sem_in
    )
    text_cp = pltpu.async_copy(text.at[pl.ds(row0, RPW)], text_all, sem_tx)
    pltpu.make_async_copy(
        cand.at[pl.ds(row0, RPW), pl.ds(0, COL0)], idx_a, sem_in
    ).wait()
    pltpu.make_async_copy(
        cand.at[pl.ds(row0, RPW), pl.ds(COL0, COL1)], idx_b, sem_in
    ).wait()

    def chunk_idx(r, off, n):
        if off < COL0:
            return idx_a.at[r, pl.ds(off, n)]
        return idx_b.at[r, pl.ds(off - COL0, n)]

    def gather_copies(r, rows_buf, sem):
        return [
            pltpu.make_async_copy(
                table.at[chunk_idx(r, off, n)],
                rows_buf.at[pl.ds(off, n)],
                sem,
            )
            for off, n in CHUNKS
        ]

    def start_gather(r, rows_buf, sem):
        for cp in gather_copies(r, rows_buf, sem):
            cp.start()

    def wait_gather(r, rows_buf, sem):
        for cp in gather_copies(r, rows_buf, sem):
            cp.wait()

    lane = lax.iota(jnp.int32, 16)
    perms = [lane ^ d for d in (1, 2, 4, 8)]

    def hsum(p):
        for pm in perms:
            p = p + jnp.take(p, pm)
        return p

    def compute(r, rows_buf, staging):
        t_vecs = [text_all[r, pl.ds(16 * k, 16)] for k in range(EMB // 16)]

        def dot16(c):
            m = [
                rows_buf[c, pl.ds(16 * k, 16)] * t_vecs[k]
                for k in range(EMB // 16)
            ]
            while len(m) > 1:
                m = [
                    m[i] + m[i + 1] for i in range(0, len(m) - 1, 2)
                ] + (m[-1:] if len(m) % 2 else [])
            return hsum(m[0])

        def group_body(g, carry):
            c0 = g * 16
            acc = jnp.zeros((16,), jnp.float32)
            for j in range(16):
                acc = jnp.where(lane == j, dot16(c0 + j), acc)
            staging[pl.ds(pl.multiple_of(g * 16, 16), 16)] = acc
            return carry

        lax.fori_loop(0, NUM_CAND // 16, group_body, 0)

        tail = (NUM_CAND // 16) * 16
        acc = jnp.zeros((16,), jnp.float32)
        for j in range(NUM_CAND - tail):
            acc = jnp.where(lane == j, dot16(tail + j), acc)
        staging[pl.ds(tail, 16)] = acc

    def out_copies(r, staging, osem):
        return [
            pltpu.make_async_copy(
                staging.at[pl.ds(0, NUM_CAND)],
                out.at[pl.ds((row0 + r) * NUM_CAND, NUM_CAND)],
                osem,
            )
        ]

    def step(r, k):
        wait_gather(r, bufs[k], gsems[k])

        @pl.when(r >= NBUF)
        def _():
            for cp in out_copies(r - NBUF, stagings[k], osems[k]):
                cp.wait()

        compute(r, bufs[k], stagings[k])
        for cp in out_copies(r, stagings[k], osems[k]):
            cp.start()

        @pl.when(r + NBUF < RPW)
        def _():
            start_gather(r + NBUF, bufs[k], gsems[k])

    for k in range(NBUF):
        start_gather(k, bufs[k], gsems[k])
    text_cp.wait()

    def round_body(i, carry):
        for k in range(NBUF):
            step(i * NBUF + k, k)
        return carry

    rounds = RPW // NBUF
    lax.fori_loop(0, rounds, round_body, 0)
    for k in range(NBUF * rounds, RPW):
        step(jnp.int32(k), k % NBUF)

    for k in range(NBUF):
        for cp in out_copies(RPW - 1, stagings[k], osems[k]):
            cp.wait()


def _rank(table, cand, text):
    mesh = plsc.VectorSubcoreMesh(core_axis_name="c", subcore_axis_name="s")
    k = pl.kernel(
        _sc_body,
        mesh=mesh,
        out_type=jax.ShapeDtypeStruct((BATCH * NUM_CAND,), jnp.float32),
        scratch_types=[
            pltpu.VMEM((RPW, COL0), jnp.int32),
            pltpu.VMEM((RPW, COL1), jnp.int32),
            pltpu.VMEM((RPW, EMB), jnp.float32),
            pltpu.VMEM((CAND_PAD, EMB), jnp.float32),
            pltpu.VMEM((CAND_PAD, EMB), jnp.float32),
            pltpu.VMEM((CAND_PAD, EMB), jnp.float32),
            pltpu.VMEM((CAND_PAD,), jnp.float32),
            pltpu.VMEM((CAND_PAD,), jnp.float32),
            pltpu.VMEM((CAND_PAD,), jnp.float32),
            pltpu.SemaphoreType.DMA,
            pltpu.SemaphoreType.DMA,
            pltpu.SemaphoreType.DMA,
            pltpu.SemaphoreType.DMA,
            pltpu.SemaphoreType.DMA,
            pltpu.SemaphoreType.DMA,
            pltpu.SemaphoreType.DMA,
            pltpu.SemaphoreType.DMA,
        ],
        compiler_params=pltpu.CompilerParams(
            needs_layout_passes=False,
            use_tc_tiling_on_sc=True,
            skip_device_barrier=True,
            disable_bounds_checks=True,
            disable_semaphore_checks=True,
        ),
    )
    return k(table, cand, text)


def kernel(pooled_output, candidate_indices, W_proj, b_proj, label_table):
    text_repr = _project(pooled_output, W_proj, b_proj)
    cand = candidate_indices.astype(jnp.int32)
    return _rank(label_table, cand, text_repr).reshape(BATCH, NUM_CAND)


# speedup vs baseline: 1.0434x; 1.0434x over previous
import functools

import jax
import jax.numpy as jnp
from jax import lax
from jax.experimental import pallas as pl
from jax.experimental.pallas import tpu as pltpu
from jax.experimental.pallas import tpu_sc as plsc

HIDDEN = 1024
EMB = 128
BATCH = 4096
NUM_CAND = 200

NUM_CORES = 2
NUM_SUBCORES = 16
NUM_WORKERS = NUM_CORES * NUM_SUBCORES
RPW = BATCH // NUM_WORKERS

IDX_CHUNKS = 2
CHUNK = NUM_CAND // IDX_CHUNKS

NUM_GROUPS = -(-NUM_CAND // 16)
CAND_PAD = NUM_GROUPS * 16

NBUF = 3


def _proj_body(p_ref, w_ref, b_ref, o_ref):
    o_ref[...] = (
        jnp.dot(p_ref[...], w_ref[...], preferred_element_type=jnp.float32)
        + b_ref[...]
    )


def _project(pooled, W, b):
    BB = 512
    return pl.pallas_call(
        _proj_body,
        grid=(BATCH // BB,),
        in_specs=[
            pl.BlockSpec((BB, HIDDEN), lambda i: (i, 0)),
            pl.BlockSpec((HIDDEN, EMB), lambda i: (0, 0)),
            pl.BlockSpec((EMB,), lambda i: (0,)),
        ],
        out_specs=pl.BlockSpec((BB, EMB), lambda i: (i, 0)),
        out_shape=jax.ShapeDtypeStruct((BATCH, EMB), jnp.float32),
    )(pooled, W, b)


def _sc_body(
    table, cand, text, out,
    idx_all, text_all, rows_0, rows_1, rows_2, st_0, st_1, st_2,
    sem_0, sem_1, sem_2, osem_0, osem_1, osem_2, sem_in, sem_tx,
):
    cid = lax.axis_index("c")
    sid = lax.axis_index("s")
    wid = sid * NUM_CORES + cid
    row0 = wid * RPW

    bufs = [rows_0, rows_1, rows_2]
    stagings = [st_0, st_1, st_2]
    gsems = [sem_0, sem_1, sem_2]
    osems = [osem_0, osem_1, osem_2]

    pltpu.async_copy(cand.at[pl.ds(row0, RPW)], idx_all, sem_in)
    text_cp = pltpu.async_copy(text.at[pl.ds(row0, RPW)], text_all, sem_tx)
    pltpu.make_async_copy(
        cand.at[pl.ds(row0, RPW)], idx_all, sem_in
    ).wait()

    def gather_copies(r, rows_buf, sem):
        return [
            pltpu.make_async_copy(
                table.at[idx_all.at[r, j]],
                rows_buf.at[pl.ds(j * CHUNK, CHUNK)],
                sem,
            )
            for j in range(IDX_CHUNKS)
        ]

    def start_gather(r, rows_buf, sem):
        for cp in gather_copies(r, rows_buf, sem):
            cp.start()

    def wait_gather(r, rows_buf, sem):
        for cp in gather_copies(r, rows_buf, sem):
            cp.wait()

    lane = lax.iota(jnp.int32, 16)
    perms = [lane ^ d for d in (1, 2, 4, 8)]

    def hsum(p):
        for pm in perms:
            p = p + jnp.take(p, pm)
        return p

    def compute(r, rows_buf, staging):
        t_vecs = [text_all[r, pl.ds(16 * k, 16)] for k in range(EMB // 16)]

        def dot16(c):
            m = [
                rows_buf[c, pl.ds(16 * k, 16)] * t_vecs[k]
                for k in range(EMB // 16)
            ]
            while len(m) > 1:
                m = [
                    m[i] + m[i + 1] for i in range(0, len(m) - 1, 2)
                ] + (m[-1:] if len(m) % 2 else [])
            return hsum(m[0])

        def group_body(g2, carry):
            for u in range(2):
                c0 = (g2 * 2 + u) * 16
                acc = jnp.zeros((16,), jnp.float32)
                for j in range(16):
                    acc = jnp.where(lane == j, dot16(c0 + j), acc)
                staging[pl.ds(pl.multiple_of(c0, 16), 16)] = acc
            return carry

        lax.fori_loop(0, NUM_CAND // 32, group_body, 0)

        tail = (NUM_CAND // 32) * 32
        acc = jnp.zeros((16,), jnp.float32)
        for j in range(NUM_CAND - tail):
            acc = jnp.where(lane == j, dot16(tail + j), acc)
        staging[pl.ds(tail, 16)] = acc

    def out_copies(r, staging, osem):
        return [
            pltpu.make_async_copy(
                staging.at[pl.ds(0, NUM_CAND)],
                out.at[pl.ds((row0 + r) * NUM_CAND, NUM_CAND)],
                osem,
            )
        ]

    def step(r, k):
        wait_gather(r, bufs[k], gsems[k])

        @pl.when(r >= NBUF)
        def _():
            for cp in out_copies(r - NBUF, stagings[k], osems[k]):
                cp.wait()

        compute(r, bufs[k], stagings[k])
        for cp in out_copies(r, stagings[k], osems[k]):
            cp.start()

        @pl.when(r + NBUF < RPW)
        def _():
            start_gather(r + NBUF, bufs[k], gsems[k])

    for k in range(NBUF):
        start_gather(k, bufs[k], gsems[k])
    text_cp.wait()

    def round_body(i, carry):
        for k in range(NBUF):
            step(i * NBUF + k, k)
        return carry

    rounds = RPW // NBUF
    lax.fori_loop(0, rounds, round_body, 0)
    for k in range(NBUF * rounds, RPW):
        step(jnp.int32(k), k % NBUF)

    for k in range(NBUF):
        for cp in out_copies(RPW - 1, stagings[k], osems[k]):
            cp.wait()


def _rank(table, cand, text):
    mesh = plsc.VectorSubcoreMesh(core_axis_name="c", subcore_axis_name="s")
    k = pl.kernel(
        _sc_body,
        mesh=mesh,
        out_type=jax.ShapeDtypeStruct((BATCH * NUM_CAND,), jnp.float32),
        scratch_types=[
            pltpu.VMEM((RPW, IDX_CHUNKS, CHUNK), jnp.int32),
            pltpu.VMEM((RPW, EMB), jnp.float32),
            pltpu.VMEM((CAND_PAD, EMB), jnp.float32),
            pltpu.VMEM((CAND_PAD, EMB), jnp.float32),
            pltpu.VMEM((CAND_PAD, EMB), jnp.float32),
            pltpu.VMEM((CAND_PAD,), jnp.float32),
            pltpu.VMEM((CAND_PAD,), jnp.float32),
            pltpu.VMEM((CAND_PAD,), jnp.float32),
            pltpu.SemaphoreType.DMA,
            pltpu.SemaphoreType.DMA,
            pltpu.SemaphoreType.DMA,
            pltpu.SemaphoreType.DMA,
            pltpu.SemaphoreType.DMA,
            pltpu.SemaphoreType.DMA,
            pltpu.SemaphoreType.DMA,
            pltpu.SemaphoreType.DMA,
        ],
        compiler_params=pltpu.CompilerParams(
            needs_layout_passes=False,
            use_tc_tiling_on_sc=True,
            skip_device_barrier=True,
            disable_bounds_checks=True,
            disable_semaphore_checks=True,
        ),
    )
    return k(table, cand, text)


def kernel(pooled_output, candidate_indices, W_proj, b_proj, label_table):
    text_repr = _project(pooled_output, W_proj, b_proj)
    cand = candidate_indices
    if cand.dtype != jnp.int32:
        cand = cand.astype(jnp.int32)
    cand3 = cand.reshape(BATCH, IDX_CHUNKS, CHUNK)
    return _rank(label_table, cand3, text_repr).reshape(BATCH, NUM_CAND)
